# initial kernel scaffold (unmeasured)
import jax
import jax.numpy as jnp
from jax import lax
from jax.experimental import pallas as pl
from jax.experimental.pallas import tpu as pltpu

N = 16
SQ = 256
D = 1024
HL = 8
DH = 128
SCALE = 0.08838834764831843
BF16 = jnp.bfloat16


def _ag_body(x_ref, out_ref, send_sems, recv_sems):
    my = lax.axis_index("i")
    out_ref[my] = x_ref[...].astype(BF16)

    sends = []
    for j in range(N - 1):
        peer = (my + 1 + j) % N
        rdma = pltpu.make_async_remote_copy(
            src_ref=out_ref.at[my],
            dst_ref=out_ref.at[my],
            send_sem=send_sems.at[j],
            recv_sem=recv_sems.at[N - 2 - j],
            device_id=(peer,),
            device_id_type=pl.DeviceIdType.MESH,
        )
        rdma.start()
        sends.append(rdma)
    for rdma in sends:
        rdma.wait_send()
    for jj in range(N - 1):
        origin = (my + 1 + jj) % N
        recv = pltpu.make_async_remote_copy(
            src_ref=out_ref.at[my],
            dst_ref=out_ref.at[origin],
            send_sem=send_sems.at[0],
            recv_sem=recv_sems.at[jj],
            device_id=(my,),
            device_id_type=pl.DeviceIdType.MESH,
        )
        recv.wait_recv()


def _compute_body(x_ref, wq_ref, wk_ref, wv_ref, wo_ref, out_ref):
    xb = x_ref[0]
    q = jnp.dot(xb, wq_ref[...], preferred_element_type=jnp.float32)
    k = jnp.dot(xb, wk_ref[...], preferred_element_type=jnp.float32)
    v = jnp.dot(xb, wv_ref[...], preferred_element_type=jnp.float32)
    qb, kb, vb = q.astype(BF16), k.astype(BF16), v.astype(BF16)
    outs = []
    for h in range(HL):
        sl = slice(h * DH, (h + 1) * DH)
        s = lax.dot_general(
            qb[:, sl], kb[:, sl], (((1,), (1,)), ((), ())),
            preferred_element_type=jnp.float32,
        ) * SCALE
        m = jnp.max(s, axis=1, keepdims=True)
        p = jnp.exp(s - m)
        l = jnp.sum(p, axis=1, keepdims=True)
        pb = (p / l).astype(BF16)
        outs.append(lax.dot_general(
            pb, vb[:, sl], (((1,), (0,)), ((), ())),
            preferred_element_type=jnp.float32,
        ))
    attn = jnp.concatenate(outs, axis=1).astype(BF16)
    out_ref[0] = jnp.dot(
        attn, wo_ref[...], preferred_element_type=jnp.float32
    ).astype(BF16)


def _rs_body(part_ref, out_ref, recv_ref, send_sems, recv_sems):
    my = lax.axis_index("i")
    sends = []
    for j in range(N - 1):
        peer = (my + 1 + j) % N
        rdma = pltpu.make_async_remote_copy(
            src_ref=part_ref.at[peer],
            dst_ref=recv_ref.at[N - 2 - j],
            send_sem=send_sems.at[j],
            recv_sem=recv_sems.at[N - 2 - j],
            device_id=(peer,),
            device_id_type=pl.DeviceIdType.MESH,
        )
        rdma.start()
        sends.append(rdma)
    for rdma in sends:
        rdma.wait_send()
    acc = part_ref[my].astype(jnp.float32)
    for jj in range(N - 1):
        recv = pltpu.make_async_remote_copy(
            src_ref=part_ref.at[my],
            dst_ref=recv_ref.at[jj],
            send_sem=send_sems.at[0],
            recv_sem=recv_sems.at[jj],
            device_id=(my,),
            device_id_type=pl.DeviceIdType.MESH,
        )
        recv.wait_recv()
        acc = acc + recv_ref[jj].astype(jnp.float32)
    out_ref[...] = acc


def kernel(x, Wq, Wo, Wk, Wv):
    x2 = x.reshape(SQ, D)
    wq, wk, wv, wo = (w.astype(BF16) for w in (Wq, Wk, Wv, Wo))

    xg = pl.pallas_call(
        _ag_body,
        out_shape=jax.ShapeDtypeStruct((N, SQ, D), BF16),
        in_specs=[pl.BlockSpec(memory_space=pltpu.VMEM)],
        out_specs=pl.BlockSpec(memory_space=pltpu.VMEM),
        scratch_shapes=[
            pltpu.SemaphoreType.DMA((N - 1,)),
            pltpu.SemaphoreType.DMA((N - 1,)),
        ],
        compiler_params=pltpu.CompilerParams(collective_id=0),
    )(x2)

    part = pl.pallas_call(
        _compute_body,
        grid=(N,),
        out_shape=jax.ShapeDtypeStruct((N, SQ, D), BF16),
        in_specs=[
            pl.BlockSpec((1, SQ, D), lambda b: (b, 0, 0)),
            pl.BlockSpec((D, HL * DH), lambda b: (0, 0)),
            pl.BlockSpec((D, HL * DH), lambda b: (0, 0)),
            pl.BlockSpec((D, HL * DH), lambda b: (0, 0)),
            pl.BlockSpec((HL * DH, D), lambda b: (0, 0)),
        ],
        out_specs=pl.BlockSpec((1, SQ, D), lambda b: (b, 0, 0)),
    )(xg, wq, wk, wv, wo)

    out = pl.pallas_call(
        _rs_body,
        out_shape=jax.ShapeDtypeStruct((SQ, D), jnp.float32),
        in_specs=[pl.BlockSpec(memory_space=pltpu.VMEM)],
        out_specs=pl.BlockSpec(memory_space=pltpu.VMEM),
        scratch_shapes=[
            pltpu.VMEM((N - 1, SQ, D), BF16),
            pltpu.SemaphoreType.DMA((N - 1,)),
            pltpu.SemaphoreType.DMA((N - 1,)),
        ],
        compiler_params=pltpu.CompilerParams(collective_id=1),
    )(part)

    return out.reshape(1, SQ, D)


# baseline (device time: 284267 ns/iter reference)
import jax
import jax.numpy as jnp
from jax import lax
from jax.experimental import pallas as pl
from jax.experimental.pallas import tpu as pltpu

N = 16
SQ = 256
D = 1024
HL = 8
DH = 128
SCALE = 0.08838834764831843
BF16 = jnp.bfloat16


def _ag_body(x_ref, out_ref, send_sems, recv_sems):
    my = lax.axis_index("i")
    out_ref[my] = x_ref[...].astype(BF16)

    sends = []
    for j in range(N - 1):
        peer = (my + 1 + j) % N
        rdma = pltpu.make_async_remote_copy(
            src_ref=out_ref.at[my],
            dst_ref=out_ref.at[my],
            send_sem=send_sems.at[j],
            recv_sem=recv_sems.at[N - 2 - j],
            device_id=(peer,),
            device_id_type=pl.DeviceIdType.MESH,
        )
        rdma.start()
        sends.append(rdma)
    for rdma in sends:
        rdma.wait_send()
    for jj in range(N - 1):
        origin = (my + 1 + jj) % N
        recv = pltpu.make_async_remote_copy(
            src_ref=out_ref.at[my],
            dst_ref=out_ref.at[origin],
            send_sem=send_sems.at[0],
            recv_sem=recv_sems.at[jj],
            device_id=(my,),
            device_id_type=pl.DeviceIdType.MESH,
        )
        recv.wait_recv()


def _compute_body(x_ref, wq_ref, wk_ref, wv_ref, wo_ref, out_ref):
    xb = x_ref[0]
    q = jnp.dot(xb, wq_ref[...], preferred_element_type=jnp.float32)
    k = jnp.dot(xb, wk_ref[...], preferred_element_type=jnp.float32)
    v = jnp.dot(xb, wv_ref[...], preferred_element_type=jnp.float32)
    qb, kb, vb = q.astype(BF16), k.astype(BF16), v.astype(BF16)
    outs = []
    for h in range(HL):
        sl = slice(h * DH, (h + 1) * DH)
        s = lax.dot_general(
            qb[:, sl], kb[:, sl], (((1,), (1,)), ((), ())),
            preferred_element_type=jnp.float32,
        ) * SCALE
        m = jnp.max(s, axis=1, keepdims=True)
        p = jnp.exp(s - m)
        l = jnp.sum(p, axis=1, keepdims=True)
        pb = (p / l).astype(BF16)
        outs.append(lax.dot_general(
            pb, vb[:, sl], (((1,), (0,)), ((), ())),
            preferred_element_type=jnp.float32,
        ))
    attn = jnp.concatenate(outs, axis=1).astype(BF16)
    out_ref[0] = jnp.dot(
        attn, wo_ref[...], preferred_element_type=jnp.float32
    ).astype(BF16)


def _rs_body(part_ref, out_ref, recv_ref, send_sems, recv_sems):
    my = lax.axis_index("i")
    sends = []
    for j in range(N - 1):
        peer = (my + 1 + j) % N
        rdma = pltpu.make_async_remote_copy(
            src_ref=part_ref.at[peer],
            dst_ref=recv_ref.at[N - 2 - j],
            send_sem=send_sems.at[j],
            recv_sem=recv_sems.at[N - 2 - j],
            device_id=(peer,),
            device_id_type=pl.DeviceIdType.MESH,
        )
        rdma.start()
        sends.append(rdma)
    for rdma in sends:
        rdma.wait_send()
    acc = part_ref[my].astype(jnp.float32)
    for jj in range(N - 1):
        recv = pltpu.make_async_remote_copy(
            src_ref=part_ref.at[my],
            dst_ref=recv_ref.at[jj],
            send_sem=send_sems.at[0],
            recv_sem=recv_sems.at[jj],
            device_id=(my,),
            device_id_type=pl.DeviceIdType.MESH,
        )
        recv.wait_recv()
        acc = acc + recv_ref[jj].astype(jnp.float32)
    out_ref[...] = acc


def kernel(x, Wq, Wo, Wk, Wv):
    x2 = x.reshape(SQ, D)
    wq, wk, wv, wo = (w.astype(BF16) for w in (Wq, Wk, Wv, Wo))

    xg = pl.pallas_call(
        _ag_body,
        out_shape=jax.ShapeDtypeStruct((N, SQ, D), BF16),
        in_specs=[pl.BlockSpec(memory_space=pltpu.VMEM)],
        out_specs=pl.BlockSpec(memory_space=pltpu.VMEM),
        scratch_shapes=[
            pltpu.SemaphoreType.DMA((N - 1,)),
            pltpu.SemaphoreType.DMA((N - 1,)),
        ],
    )(x2)

    part = pl.pallas_call(
        _compute_body,
        grid=(N,),
        out_shape=jax.ShapeDtypeStruct((N, SQ, D), BF16),
        in_specs=[
            pl.BlockSpec((1, SQ, D), lambda b: (b, 0, 0)),
            pl.BlockSpec((D, HL * DH), lambda b: (0, 0)),
            pl.BlockSpec((D, HL * DH), lambda b: (0, 0)),
            pl.BlockSpec((D, HL * DH), lambda b: (0, 0)),
            pl.BlockSpec((HL * DH, D), lambda b: (0, 0)),
        ],
        out_specs=pl.BlockSpec((1, SQ, D), lambda b: (b, 0, 0)),
    )(xg, wq, wk, wv, wo)

    out = pl.pallas_call(
        _rs_body,
        out_shape=jax.ShapeDtypeStruct((SQ, D), jnp.float32),
        in_specs=[pl.BlockSpec(memory_space=pltpu.VMEM)],
        out_specs=pl.BlockSpec(memory_space=pltpu.VMEM),
        scratch_shapes=[
            pltpu.VMEM((N - 1, SQ, D), BF16),
            pltpu.SemaphoreType.DMA((N - 1,)),
            pltpu.SemaphoreType.DMA((N - 1,)),
        ],
    )(part)

    return out.reshape(1, SQ, D)


# device time: 143578 ns/iter; 1.9799x vs baseline; 1.9799x over previous
import jax
import jax.numpy as jnp
from jax import lax
from jax.experimental import pallas as pl
from jax.experimental.pallas import tpu as pltpu

N = 16
NZ = 4
NP = 4
SQ = 256
D = 1024
HL = 8
DH = 128
SCALE = 0.08838834764831843
BF16 = jnp.bfloat16
F32 = jnp.float32


def _body(
    x_ref, wq_ref, wk_ref, wv_ref, wo_ref, out_ref,
    xg, ps, part_send, rsp_recv_buf, ps_send, rsz_recv_buf,
    col_send, col_recv, plane_send, plane_recv,
    rsp_send, rsp_recv, rsz_send, rsz_recv,
):
    my = lax.axis_index("i")
    myz = my // NP
    myp = my % NP

    def _rdma(src, dst, ssem, rsem, tgt):
        return pltpu.make_async_remote_copy(
            src_ref=src, dst_ref=dst, send_sem=ssem, recv_sem=rsem,
            device_id=(tgt,), device_id_type=pl.DeviceIdType.MESH,
        )

    def _partial(xb):
        q = jnp.dot(xb, wq_ref[...], preferred_element_type=F32)
        k = jnp.dot(xb, wk_ref[...], preferred_element_type=F32)
        v = jnp.dot(xb, wv_ref[...], preferred_element_type=F32)
        qb, kb, vb = q.astype(BF16), k.astype(BF16), v.astype(BF16)
        outs = []
        for h in range(HL):
            sl = slice(h * DH, (h + 1) * DH)
            s = lax.dot_general(
                qb[:, sl], kb[:, sl], (((1,), (1,)), ((), ())),
                preferred_element_type=F32,
            ) * SCALE
            m = jnp.max(s, axis=1, keepdims=True)
            p = jnp.exp(s - m)
            l = jnp.sum(p, axis=1, keepdims=True)
            pb = (p / l).astype(BF16)
            outs.append(lax.dot_general(
                pb, vb[:, sl], (((1,), (0,)), ((), ())),
                preferred_element_type=F32,
            ))
        attn = jnp.concatenate(outs, axis=1).astype(BF16)
        return jnp.dot(attn, wo_ref[...], preferred_element_type=F32)

    xg[my] = x_ref[...].astype(BF16)
    for k in (1, 2, 3):
        tgt = NP * ((myz + k) % NZ) + myp
        _rdma(xg.at[my], xg.at[my],
              col_send.at[k - 1], col_recv.at[NZ - k], tgt).start()
    for j in range(3):
        tgt = NP * myz + (myp + 1 + j) % NP
        _rdma(xg.at[my], xg.at[my],
              plane_send.at[0, j], plane_recv.at[0, 2 - j], tgt).start()

    ps[0] = _partial(xg[my])

    for o in (1, 2, 3):
        b = NP * ((myz + o) % NZ) + myp
        _rdma(xg.at[my], xg.at[b], col_send.at[0], col_recv.at[o],
              my).wait_recv()
        for j in range(3):
            tgt = NP * myz + (myp + 1 + j) % NP
            _rdma(xg.at[b], xg.at[b],
                  plane_send.at[o, j], plane_recv.at[o, 2 - j], tgt).start()
        ps[o] = _partial(xg[b])

    for j in range(3):
        agg = NP * myz + (myp + 1 + j) % NP
        for o in range(4):
            b = NP * ((myz + o) % NZ) + (myp + 1 + j) % NP
            _rdma(xg.at[my], xg.at[b], col_send.at[0], plane_recv.at[o, j],
                  my).wait_recv()
            part_send[o, j] = _partial(xg[b]).astype(BF16)
            _rdma(part_send.at[o, j], rsp_recv_buf.at[o, 2 - j],
                  rsp_send.at[o, j], rsp_recv.at[o, 2 - j], agg).start()

    for o in range(4):
        acc = ps[o]
        for j in range(3):
            _rdma(xg.at[my], rsp_recv_buf.at[o, j], col_send.at[0],
                  rsp_recv.at[o, j], my).wait_recv()
            acc = acc + rsp_recv_buf[o, j].astype(F32)
        if o == 0:
            ps[0] = acc
        else:
            ps_send[o - 1] = acc.astype(BF16)
            tgt = NP * ((myz + o) % NZ) + myp
            _rdma(ps_send.at[o - 1], rsz_recv_buf.at[3 - o],
                  rsz_send.at[o - 1], rsz_recv.at[NZ - o], tgt).start()

    acc = ps[0]
    for orr in (1, 2, 3):
        _rdma(xg.at[my], rsz_recv_buf.at[orr - 1], col_send.at[0],
              rsz_recv.at[orr], my).wait_recv()
        acc = acc + rsz_recv_buf[orr - 1].astype(F32)
    out_ref[...] = acc

    for k in (1, 2, 3):
        _rdma(xg.at[my], xg.at[my], col_send.at[k - 1], col_recv.at[0],
              my).wait_send()
    for o in range(4):
        for j in range(3):
            _rdma(xg.at[my], xg.at[my], plane_send.at[o, j],
                  plane_recv.at[0, 0], my).wait_send()
            _rdma(part_send.at[o, j], rsp_recv_buf.at[o, j],
                  rsp_send.at[o, j], rsp_recv.at[0, 0], my).wait_send()
    for o in (1, 2, 3):
        _rdma(ps_send.at[o - 1], rsz_recv_buf.at[o - 1],
              rsz_send.at[o - 1], rsz_recv.at[0], my).wait_send()


def kernel(x, Wq, Wo, Wk, Wv):
    x2 = x.reshape(SQ, D)
    wq, wk, wv, wo = (w.astype(BF16) for w in (Wq, Wk, Wv, Wo))

    out = pl.pallas_call(
        _body,
        out_shape=jax.ShapeDtypeStruct((SQ, D), F32),
        in_specs=[pl.BlockSpec(memory_space=pltpu.VMEM)] * 5,
        out_specs=pl.BlockSpec(memory_space=pltpu.VMEM),
        scratch_shapes=[
            pltpu.VMEM((N, SQ, D), BF16),
            pltpu.VMEM((NZ, SQ, D), F32),
            pltpu.VMEM((NZ, 3, SQ, D), BF16),
            pltpu.VMEM((NZ, 3, SQ, D), BF16),
            pltpu.VMEM((3, SQ, D), BF16),
            pltpu.VMEM((3, SQ, D), BF16),
            pltpu.SemaphoreType.DMA((3,)),
            pltpu.SemaphoreType.DMA((NZ,)),
            pltpu.SemaphoreType.DMA((NZ, 3)),
            pltpu.SemaphoreType.DMA((NZ, 3)),
            pltpu.SemaphoreType.DMA((NZ, 3)),
            pltpu.SemaphoreType.DMA((NZ, 3)),
            pltpu.SemaphoreType.DMA((3,)),
            pltpu.SemaphoreType.DMA((NZ,)),
        ],
    )(x2, wq, wk, wv, wo)

    return out.reshape(1, SQ, D)
